# baseline (speedup 1.0000x reference)
import functools

import jax
import jax.numpy as jnp
import numpy as np
from jax import lax
from jax.experimental import pallas as pl
from jax.experimental.pallas import tpu as pltpu
from jax.experimental.pallas import tpu_sc as plsc

_N = 2_000_000
_NW = 32
_NBINS = 15
_UNROLL = 8
_CBITS = 13
_ONE_ACC = (1 << _CBITS) + 1

_ROWS_SC = 81920
_RPW = _ROWS_SC // _NW
_CHUNK = 640
_N_CHUNKS = _RPW // _CHUNK

_TC_ROW0 = 10240
_TC_ROWS_TOTAL = _N // 128
_TC_BLOCK = 1024
_TC_GRID = -(-(_TC_ROWS_TOTAL - _TC_ROW0) // _TC_BLOCK)

_BOUNDS = [float(v) for v in np.arange(16, dtype=np.float32)
           * np.float32(1.0 / 15.0)]

_MESH = plsc.VectorSubcoreMesh(core_axis_name="c", subcore_axis_name="s")


@functools.partial(
    pl.kernel,
    out_type=jax.ShapeDtypeStruct((_NW, 45, 16), jnp.float32),
    mesh=_MESH,
    compiler_params=pltpu.CompilerParams(needs_layout_passes=False),
    scratch_types=[
        pltpu.VMEM((_CHUNK * 16,), jnp.float32),
        pltpu.VMEM((_CHUNK * 16,), jnp.float32),
        pltpu.VMEM((_CHUNK * 16,), jnp.int32),
        pltpu.VMEM((_CHUNK * 16,), jnp.int32),
        pltpu.VMEM((_CHUNK * 16,), jnp.int32),
        pltpu.VMEM((_CHUNK * 16,), jnp.int32),
        pltpu.VMEM((272,), jnp.int32),
        pltpu.VMEM((272,), jnp.float32),
        pltpu.VMEM((45, 16), jnp.float32),
        pltpu.SemaphoreType.DMA,
        pltpu.SemaphoreType.DMA,
    ],
)
def _sc_ece(conf_hbm, pred_hbm, lab_hbm, out_hbm,
            cbuf0, cbuf1, pbuf0, pbuf1, lbuf0, lbuf1,
            acci, accs, sbuf, sem0, sem1):
    bufs = ((cbuf0, pbuf0, lbuf0), (cbuf1, pbuf1, lbuf1))
    wid = lax.axis_index("s") * 2 + lax.axis_index("c")
    base = wid * (_RPW * 16)

    zi = jnp.zeros((16,), jnp.int32)
    zf = jnp.zeros((16,), jnp.float32)
    for i in range(17):
        acci[pl.ds(i * 16, 16)] = zi
        accs[pl.ds(i * 16, 16)] = zf

    lane = lax.iota(jnp.int32, 16)
    lane16 = lane + 16
    delta = jnp.float32(1.0 / 15.0)
    sems = (sem0, sem1)

    def issue(k):
        b = k % 2
        cb, pb, lb = bufs[b]
        sl = pl.ds(base + k * (_CHUNK * 16), _CHUNK * 16)
        return (pltpu.async_copy(conf_hbm.at[sl], cb, sems[b]),
                pltpu.async_copy(pred_hbm.at[sl], pb, sems[b]),
                pltpu.async_copy(lab_hbm.at[sl], lb, sems[b]))

    def compute(k):
        cb, pb, lb = bufs[k % 2]

        @plsc.parallel_loop(0, _CHUNK, unroll=_UNROLL)
        def body(i):
            off = i * 16
            conf = cb[pl.ds(off, 16)]
            pred = pb[pl.ds(off, 16)]
            labv = lb[pl.ds(off, 16)]
            packed = jnp.where(pred == labv, _ONE_ACC, 1)
            b0 = (conf * 15.0 + 0.5).astype(jnp.int32)
            bound = b0.astype(jnp.float32) * delta
            adj = jnp.where(conf <= bound, lane, lane16)
            idx = b0 * 16 + adj
            plsc.addupdate_scatter(acci, [idx], packed)
            plsc.addupdate_scatter(accs, [idx], conf)

    copies = issue(0)
    for k in range(_N_CHUNKS):
        for c in copies:
            c.wait()
        if k + 1 < _N_CHUNKS:
            copies = issue(k + 1)
        compute(k)

    for bb in range(_NBINS):
        packed = acci[pl.ds((bb + 1) * 16, 16)]
        sbuf[bb] = (packed & ((1 << _CBITS) - 1)).astype(jnp.float32)
        sbuf[15 + bb] = accs[pl.ds((bb + 1) * 16, 16)]
        sbuf[30 + bb] = (packed >> _CBITS).astype(jnp.float32)
    pltpu.sync_copy(sbuf, out_hbm.at[wid])


def _tc_hist(c_ref, p_ref, l_ref, o_ref):
    step = pl.program_id(0)

    @pl.when(step == 0)
    def _():
        o_ref[...] = jnp.zeros_like(o_ref)

    conf = c_ref[...]
    acc = jnp.where(p_ref[...] == l_ref[...], 1.0, 0.0)
    row = (_TC_ROW0 + step * _TC_BLOCK
           + lax.broadcasted_iota(jnp.int32, conf.shape, 0))
    valid = row < _TC_ROWS_TOTAL
    b0 = (conf * 15.0 + 0.5).astype(jnp.int32)
    bound = b0.astype(jnp.float32) * jnp.float32(1.0 / 15.0)
    bi = b0 - jnp.where(conf <= bound, 1, 0)
    bi = jnp.where(valid, bi, -1)
    for bb in range(_NBINS):
        m = bi == bb
        mf = jnp.where(m, 1.0, 0.0)
        o_ref[bb, :] += jnp.sum(mf, axis=0)
        o_ref[15 + bb, :] += jnp.sum(jnp.where(m, conf, 0.0), axis=0)
        o_ref[30 + bb, :] += jnp.sum(mf * acc, axis=0)


def _tc_combine(p_ref, t_ref, o_ref):
    q = jnp.sum(p_ref[...], axis=0)
    s = jnp.sum(q, axis=1) + jnp.sum(t_ref[...], axis=1)
    count = s[0:15]
    sumc = s[15:30]
    suma = s[30:45]
    safe = jnp.maximum(count, 1.0)
    prop = count / float(_N)
    term = jnp.where(count > 0.0,
                     jnp.abs(sumc / safe - suma / safe) * prop, 0.0)
    o_ref[...] = jnp.sum(term).reshape(1, 1)


def kernel(confidences, predictions, labels):
    sc_partials = _sc_ece(confidences, predictions, labels)
    conf2 = confidences.reshape(_TC_ROWS_TOTAL, 128)
    pred2 = predictions.reshape(_TC_ROWS_TOTAL, 128)
    lab2 = labels.reshape(_TC_ROWS_TOTAL, 128)
    bspec = pl.BlockSpec((_TC_BLOCK, 128),
                         lambda i: (_TC_ROW0 // _TC_BLOCK + i, 0))
    tc_partials = pl.pallas_call(
        _tc_hist,
        grid=(_TC_GRID,),
        in_specs=[bspec, bspec, bspec],
        out_specs=pl.BlockSpec((45, 128), lambda i: (0, 0)),
        out_shape=jax.ShapeDtypeStruct((45, 128), jnp.float32),
    )(conf2, pred2, lab2)
    ece = pl.pallas_call(
        _tc_combine,
        out_shape=jax.ShapeDtypeStruct((1, 1), jnp.float32),
    )(sc_partials, tc_partials)
    return ece.reshape((1,))


# speedup vs baseline: 1.0305x; 1.0305x over previous
import functools

import jax
import jax.numpy as jnp
import numpy as np
from jax import lax
from jax.experimental import pallas as pl
from jax.experimental.pallas import tpu as pltpu
from jax.experimental.pallas import tpu_sc as plsc

_N = 2_000_000
_NW = 32
_NBINS = 15
_UNROLL = 8
_CBITS = 13
_ONE_ACC = (1 << _CBITS) + 1

_ROWS_SC = 81920
_RPW = _ROWS_SC // _NW
_CHUNK = 640
_N_CHUNKS = _RPW // _CHUNK

_TC_ROW0 = 10240
_TC_ROWS_TOTAL = _N // 128
_TC_BLOCK = 512
_TC_GRID = -(-(_TC_ROWS_TOTAL - _TC_ROW0) // _TC_BLOCK)

_BOUNDS = [float(v) for v in np.arange(16, dtype=np.float32)
           * np.float32(1.0 / 15.0)]

_MESH = plsc.VectorSubcoreMesh(core_axis_name="c", subcore_axis_name="s")


@functools.partial(
    pl.kernel,
    out_type=jax.ShapeDtypeStruct((_NW, 45, 16), jnp.float32),
    mesh=_MESH,
    compiler_params=pltpu.CompilerParams(needs_layout_passes=False),
    scratch_types=[
        pltpu.VMEM((_CHUNK * 16,), jnp.float32),
        pltpu.VMEM((_CHUNK * 16,), jnp.float32),
        pltpu.VMEM((_CHUNK * 16,), jnp.int32),
        pltpu.VMEM((_CHUNK * 16,), jnp.int32),
        pltpu.VMEM((_CHUNK * 16,), jnp.int32),
        pltpu.VMEM((_CHUNK * 16,), jnp.int32),
        pltpu.VMEM((272,), jnp.int32),
        pltpu.VMEM((272,), jnp.float32),
        pltpu.VMEM((45, 16), jnp.float32),
        pltpu.SemaphoreType.DMA,
        pltpu.SemaphoreType.DMA,
    ],
)
def _sc_ece(conf_hbm, pred_hbm, lab_hbm, out_hbm,
            cbuf0, cbuf1, pbuf0, pbuf1, lbuf0, lbuf1,
            acci, accs, sbuf, sem0, sem1):
    bufs = ((cbuf0, pbuf0, lbuf0), (cbuf1, pbuf1, lbuf1))
    wid = lax.axis_index("s") * 2 + lax.axis_index("c")
    base = wid * (_RPW * 16)

    zi = jnp.zeros((16,), jnp.int32)
    zf = jnp.zeros((16,), jnp.float32)
    for i in range(17):
        acci[pl.ds(i * 16, 16)] = zi
        accs[pl.ds(i * 16, 16)] = zf

    lane = lax.iota(jnp.int32, 16)
    lane16 = lane + 16
    delta = jnp.float32(1.0 / 15.0)
    sems = (sem0, sem1)

    def issue(k):
        b = k % 2
        cb, pb, lb = bufs[b]
        sl = pl.ds(base + k * (_CHUNK * 16), _CHUNK * 16)
        return (pltpu.async_copy(conf_hbm.at[sl], cb, sems[b]),
                pltpu.async_copy(pred_hbm.at[sl], pb, sems[b]),
                pltpu.async_copy(lab_hbm.at[sl], lb, sems[b]))

    def compute(k):
        cb, pb, lb = bufs[k % 2]

        @plsc.parallel_loop(0, _CHUNK, unroll=_UNROLL)
        def body(i):
            off = i * 16
            conf = cb[pl.ds(off, 16)]
            pred = pb[pl.ds(off, 16)]
            labv = lb[pl.ds(off, 16)]
            packed = jnp.where(pred == labv, _ONE_ACC, 1)
            b0 = (conf * 15.0 + 0.5).astype(jnp.int32)
            bound = b0.astype(jnp.float32) * delta
            adj = jnp.where(conf <= bound, lane, lane16)
            idx = b0 * 16 + adj
            plsc.addupdate_scatter(acci, [idx], packed)
            plsc.addupdate_scatter(accs, [idx], conf)

    copies = issue(0)
    for k in range(_N_CHUNKS):
        for c in copies:
            c.wait()
        if k + 1 < _N_CHUNKS:
            copies = issue(k + 1)
        compute(k)

    for bb in range(_NBINS):
        packed = acci[pl.ds((bb + 1) * 16, 16)]
        sbuf[bb] = (packed & ((1 << _CBITS) - 1)).astype(jnp.float32)
        sbuf[15 + bb] = accs[pl.ds((bb + 1) * 16, 16)]
        sbuf[30 + bb] = (packed >> _CBITS).astype(jnp.float32)
    pltpu.sync_copy(sbuf, out_hbm.at[wid])


def _tc_hist(c_ref, p_ref, l_ref, o_ref):
    step = pl.program_id(0)

    @pl.when(step == 0)
    def _():
        o_ref[...] = jnp.zeros_like(o_ref)

    conf = c_ref[...]
    acc = jnp.where(p_ref[...] == l_ref[...], 1.0, 0.0)
    row = (_TC_ROW0 + step * _TC_BLOCK
           + lax.broadcasted_iota(jnp.int32, conf.shape, 0))
    valid = row < _TC_ROWS_TOTAL
    b0 = (conf * 15.0 + 0.5).astype(jnp.int32)
    bound = b0.astype(jnp.float32) * jnp.float32(1.0 / 15.0)
    bi = b0 - jnp.where(conf <= bound, 1, 0)
    bi = jnp.where(valid, bi, -1)
    for bb in range(_NBINS):
        m = bi == bb
        mf = jnp.where(m, 1.0, 0.0)
        o_ref[bb, :] += jnp.sum(mf, axis=0)
        o_ref[15 + bb, :] += jnp.sum(jnp.where(m, conf, 0.0), axis=0)
        o_ref[30 + bb, :] += jnp.sum(mf * acc, axis=0)


def _tc_combine(p_ref, t_ref, o_ref):
    q = jnp.sum(p_ref[...], axis=0)
    s = jnp.sum(q, axis=1) + jnp.sum(t_ref[...], axis=1)
    count = s[0:15]
    sumc = s[15:30]
    suma = s[30:45]
    safe = jnp.maximum(count, 1.0)
    prop = count / float(_N)
    term = jnp.where(count > 0.0,
                     jnp.abs(sumc / safe - suma / safe) * prop, 0.0)
    o_ref[...] = jnp.sum(term).reshape(1, 1)


def kernel(confidences, predictions, labels):
    sc_partials = _sc_ece(confidences, predictions, labels)
    conf2 = confidences.reshape(_TC_ROWS_TOTAL, 128)
    pred2 = predictions.reshape(_TC_ROWS_TOTAL, 128)
    lab2 = labels.reshape(_TC_ROWS_TOTAL, 128)
    bspec = pl.BlockSpec((_TC_BLOCK, 128),
                         lambda i: (_TC_ROW0 // _TC_BLOCK + i, 0))
    tc_partials = pl.pallas_call(
        _tc_hist,
        grid=(_TC_GRID,),
        in_specs=[bspec, bspec, bspec],
        out_specs=pl.BlockSpec((45, 128), lambda i: (0, 0)),
        out_shape=jax.ShapeDtypeStruct((45, 128), jnp.float32),
    )(conf2, pred2, lab2)
    ece = pl.pallas_call(
        _tc_combine,
        out_shape=jax.ShapeDtypeStruct((1, 1), jnp.float32),
    )(sc_partials, tc_partials)
    return ece.reshape((1,))
